# Initial kernel scaffold; baseline (speedup 1.0000x reference)
import jax
import jax.numpy as jnp
from jax.experimental import pallas as pl


def kernel(x, edge_index, W_enc, b_enc, W_rel, b_rel, W_root, ln_w, ln_b, gate_W, gate_b, exp_W1, exp_b1, exp_W2, exp_b2):
    raise NotImplementedError("write your pallas kernel here")


# speedup vs baseline: 7.2643x; 7.2643x over previous
import functools
import math

import jax
import jax.numpy as jnp
import numpy as np
from jax import lax
from jax.experimental import pallas as pl
from jax.experimental.pallas import tpu as pltpu
from jax.experimental.pallas import tpu_sc as plsc

N = 10000
E = 320000
D = 128

_NC = 2
_NS = 16
_NW = _NC * _NS
_EPT = E // _NW
_CH = 125
_NK = _EPT // _CH
_RPT = 624
_ZR = 104
_TAIL = N - _NS * _RPT


def _sc_aggregate_body(h_hbm, src_hbm, dst_hbm, out_hbm,
                       src_v, dst_v, rows_v, zbuf, agg_sh, sem):
    c = lax.axis_index("c")
    s = lax.axis_index("s")
    w = c * _NS + s

    z16 = jnp.zeros((16,), jnp.float32)

    def zero_body(i, carry):
        for j in range(8):
            zbuf[i, pl.ds(j * 16, 16)] = z16
        return carry

    lax.fori_loop(0, _ZR, zero_body, 0)
    row0 = s * _RPT
    for j in range(_RPT // _ZR):
        pltpu.sync_copy(zbuf, agg_sh.at[pl.ds(row0 + j * _ZR, _ZR)])

    @pl.when(s == _NS - 1)
    def _():
        pltpu.sync_copy(zbuf.at[pl.ds(0, _TAIL)],
                        agg_sh.at[pl.ds(_NS * _RPT, _TAIL)])

    plsc.subcore_barrier()

    erow = w * _NK
    pltpu.sync_copy(src_hbm.at[pl.ds(erow, _NK)], src_v)
    pltpu.sync_copy(dst_hbm.at[pl.ds(erow, _NK)], dst_v)

    def edge_body(k, carry):
        pltpu.async_copy(h_hbm.at[src_v.at[k]], rows_v, sem).wait()
        pltpu.sync_copy(rows_v, agg_sh.at[dst_v.at[k]], add=True)
        return carry

    lax.fori_loop(0, _NK, edge_body, 0)
    plsc.subcore_barrier()

    for j in range(_RPT // _ZR):
        r = row0 + j * _ZR
        pltpu.sync_copy(agg_sh.at[pl.ds(r, _ZR)], zbuf)
        pltpu.sync_copy(zbuf, out_hbm.at[c, pl.ds(r, _ZR)])

    @pl.when(s == _NS - 1)
    def _():
        r = _NS * _RPT
        pltpu.sync_copy(agg_sh.at[pl.ds(r, _TAIL)], zbuf.at[pl.ds(0, _TAIL)])
        pltpu.sync_copy(zbuf.at[pl.ds(0, _TAIL)], out_hbm.at[c, pl.ds(r, _TAIL)])


@functools.partial(
    pl.kernel,
    out_type=jax.ShapeDtypeStruct((_NC, N, D), jnp.float32),
    mesh=plsc.VectorSubcoreMesh(core_axis_name="c", subcore_axis_name="s"),
    scratch_types=[
        pltpu.VMEM((_NK, _CH), jnp.int32),
        pltpu.VMEM((_NK, _CH), jnp.int32),
        pltpu.VMEM((_CH, D), jnp.float32),
        pltpu.VMEM((_ZR, D), jnp.float32),
        pltpu.VMEM_SHARED((N, D), jnp.float32),
        pltpu.SemaphoreType.DMA,
    ],
)
def _sc_aggregate(h_hbm, src_hbm, dst_hbm, out_hbm,
                  src_v, dst_v, rows_v, zbuf, agg_sh, sem):
    _sc_aggregate_body(h_hbm, src_hbm, dst_hbm, out_hbm,
                       src_v, dst_v, rows_v, zbuf, agg_sh, sem)


_BLK = 1000
_SQRT2 = math.sqrt(2.0)


def _gelu(t):
    return t * 0.5 * (lax.erf(t / _SQRT2) + 1.0)


def _enc_body(x_ref, w_ref, b_ref, o_ref):
    o_ref[...] = _gelu(
        jnp.dot(x_ref[...], w_ref[...], preferred_element_type=jnp.float32)
        + b_ref[...]
    )


def _encoder(x, W_enc, b_enc):
    return pl.pallas_call(
        _enc_body,
        grid=(N // _BLK,),
        in_specs=[
            pl.BlockSpec((_BLK, D), lambda i: (i, 0)),
            pl.BlockSpec((D, D), lambda i: (0, 0)),
            pl.BlockSpec((1, D), lambda i: (0, 0)),
        ],
        out_specs=pl.BlockSpec((_BLK, D), lambda i: (i, 0)),
        out_shape=jax.ShapeDtypeStruct((N, D), jnp.float32),
    )(x, W_enc, b_enc.reshape(1, D))


def _layer_body(h_ref, p0_ref, p1_ref, wrel_ref, brel_ref, wroot_ref,
                lnw_ref, lnb_ref, o_ref):
    h = h_ref[...]
    agg = p0_ref[...] + p1_ref[...]
    z = (jnp.dot(agg, wrel_ref[...], preferred_element_type=jnp.float32)
         + jnp.dot(h, wroot_ref[...], preferred_element_type=jnp.float32)
         + brel_ref[...])
    h2 = _gelu(z) + h
    mu = jnp.mean(h2, axis=-1, keepdims=True)
    d = h2 - mu
    var = jnp.mean(d * d, axis=-1, keepdims=True)
    o_ref[...] = d * lax.rsqrt(var + 1e-5) * lnw_ref[...] + lnb_ref[...]


def _layer(h, p0, p1, wrel, brel, wroot, lnw, lnb):
    return pl.pallas_call(
        _layer_body,
        grid=(N // _BLK,),
        in_specs=[
            pl.BlockSpec((_BLK, D), lambda i: (i, 0)),
            pl.BlockSpec((_BLK, D), lambda i: (i, 0)),
            pl.BlockSpec((_BLK, D), lambda i: (i, 0)),
            pl.BlockSpec((D, D), lambda i: (0, 0)),
            pl.BlockSpec((1, D), lambda i: (0, 0)),
            pl.BlockSpec((D, D), lambda i: (0, 0)),
            pl.BlockSpec((1, D), lambda i: (0, 0)),
            pl.BlockSpec((1, D), lambda i: (0, 0)),
        ],
        out_specs=pl.BlockSpec((_BLK, D), lambda i: (i, 0)),
        out_shape=jax.ShapeDtypeStruct((N, D), jnp.float32),
    )(h, p0, p1, wrel, brel.reshape(1, D), wroot,
      lnw.reshape(1, D), lnb.reshape(1, D))


def _dec_body(h_ref, gw_ref, gb_ref, w1_ref, b1_ref, w2_ref, b2_ref,
              m16_ref, msel_ref, o_ref):
    h = h_ref[...]
    logits = jnp.dot(h, gw_ref[...], preferred_element_type=jnp.float32) + gb_ref[...]
    m = jnp.max(logits, axis=-1, keepdims=True)
    p = jnp.exp(logits - m)
    denom = jnp.dot(p, m16_ref[...], preferred_element_type=jnp.float32)
    gates = p / denom
    hid = _gelu(
        jnp.dot(h, w1_ref[...], preferred_element_type=jnp.float32) + b1_ref[...]
    )
    head = jnp.dot(hid, w2_ref[...], preferred_element_type=jnp.float32) + b2_ref[...]
    o_ref[...] = jnp.dot(gates * head, msel_ref[...],
                         preferred_element_type=jnp.float32)


def _decoder(h, gw, gb, w1, b1, w2, b2, m16, msel, T, TH):
    return pl.pallas_call(
        _dec_body,
        grid=(N // _BLK,),
        in_specs=[
            pl.BlockSpec((_BLK, D), lambda i: (i, 0)),
            pl.BlockSpec((D, TH), lambda i: (0, 0)),
            pl.BlockSpec((1, TH), lambda i: (0, 0)),
            pl.BlockSpec((D, TH * D), lambda i: (0, 0)),
            pl.BlockSpec((1, TH * D), lambda i: (0, 0)),
            pl.BlockSpec((TH * D, TH), lambda i: (0, 0)),
            pl.BlockSpec((1, TH), lambda i: (0, 0)),
            pl.BlockSpec((TH, TH), lambda i: (0, 0)),
            pl.BlockSpec((TH, T), lambda i: (0, 0)),
        ],
        out_specs=pl.BlockSpec((_BLK, T), lambda i: (i, 0)),
        out_shape=jax.ShapeDtypeStruct((N, T), jnp.float32),
    )(h, gw, gb, w1, b1, w2, b2, m16, msel)


def kernel(x, edge_index, W_enc, b_enc, W_rel, b_rel, W_root, ln_w, ln_b,
           gate_W, gate_b, exp_W1, exp_b1, exp_W2, exp_b2):
    L = W_rel.shape[0]
    T, Dh, H = gate_W.shape
    TH = T * H

    src2d = edge_index[0].reshape(E // _CH, _CH)
    dst2d = edge_index[1].reshape(E // _CH, _CH)

    h = _encoder(x, W_enc, b_enc)
    for l in range(L):
        partials = _sc_aggregate(h, src2d, dst2d)
        h = _layer(h, partials[0], partials[1], W_rel[l], b_rel[l],
                   W_root[l], ln_w[l], ln_b[l])

    gw = jnp.transpose(gate_W, (1, 0, 2)).reshape(D, TH)
    gb = gate_b.reshape(1, TH)
    w1 = jnp.transpose(exp_W1, (2, 0, 1, 3)).reshape(D, TH * D)
    b1 = exp_b1.reshape(1, TH * D)
    w2flat = exp_W2[..., 0].reshape(TH * D)
    blockmask = jnp.kron(jnp.eye(TH, dtype=jnp.float32),
                         jnp.ones((D, 1), jnp.float32))
    w2 = w2flat[:, None] * blockmask
    b2 = exp_b2.reshape(1, TH)
    m16 = jnp.kron(jnp.eye(T, dtype=jnp.float32),
                   jnp.ones((H, H), jnp.float32))
    msel = jnp.kron(jnp.eye(T, dtype=jnp.float32),
                    jnp.ones((H, 1), jnp.float32))
    return _decoder(h, gw, gb, w1, b1, w2, b2, m16, msel, T, TH)
